# baseline (speedup 1.0000x reference)
import functools
import math

import jax
import jax.numpy as jnp
from jax import lax
from jax.experimental import pallas as pl
from jax.experimental.pallas import tpu as pltpu
from jax.experimental.pallas import tpu_sc as plsc

PAD_IDX = 0
LANES = 16


@functools.partial(jax.jit, static_argnames=("granule", "nbuf"))
def _sc_embed(idx, table2, granule, nbuf):
    n_workers, n_gran, _ = idx.shape
    N = n_workers * n_gran * granule
    D = table2.shape[1] // 2
    scale = math.sqrt(float(D))
    n_per_w = n_gran * granule
    mesh = plsc.VectorSubcoreMesh(core_axis_name="c", subcore_axis_name="s")
    info = plsc.get_sparse_core_info()
    nc = info.num_cores

    @functools.partial(
        pl.kernel,
        mesh=mesh,
        out_type=jax.ShapeDtypeStruct((N // 2, 2 * D), jnp.float32),
        compiler_params=pltpu.CompilerParams(use_tc_tiling_on_sc=True),
        scratch_types=[
            pltpu.VMEM((n_gran, granule), jnp.int32),
            pltpu.VMEM((nbuf, granule), jnp.int32),
            pltpu.VMEM((nbuf, granule, 2 * D), jnp.float32),
            pltpu.VMEM((nbuf, granule // 2, 2 * D), jnp.float32),
            pltpu.SemaphoreType.DMA((nbuf,)),
            pltpu.SemaphoreType.DMA((nbuf,)),
        ],
    )
    def lookup(idx_hbm, tbl_hbm, out_hbm, idx_v, pidx_v, rows_v, out_v,
               gsem, osem):
        wid = lax.axis_index("s") * nc + lax.axis_index("c")
        wbase = wid * n_per_w

        def fire_gather(g, b):
            def pair_body(q, carry):
                iv = idx_v[g, pl.ds(q * LANES, LANES)]
                pidx_v[b, pl.ds(q * LANES, LANES)] = jnp.right_shift(iv, 1)
                return carry
            lax.fori_loop(0, granule // LANES, pair_body, 0)
            pltpu.make_async_copy(
                tbl_hbm.at[pidx_v.at[b]], rows_v.at[b], gsem.at[b]
            ).start()

        def wait_gather(b):
            pltpu.make_async_copy(
                tbl_hbm.at[pidx_v.at[b]], rows_v.at[b], gsem.at[b]
            ).wait()

        def fire_scatter(g, b):
            off = pl.multiple_of((wbase + g * granule) // 2, granule // 2)
            pltpu.make_async_copy(
                out_v.at[b],
                out_hbm.at[pl.ds(off, granule // 2)],
                osem.at[b],
            ).start()

        def wait_scatter(b):
            off = pl.multiple_of(wbase // 2, granule // 2)
            pltpu.make_async_copy(
                out_v.at[b], out_hbm.at[pl.ds(off, granule // 2)],
                osem.at[b]
            ).wait()

        def compute(g, b):
            def grp_body(q, carry):
                iv = idx_v[g, pl.ds(q * LANES, LANES)]
                sv = jnp.where(iv != PAD_IDX, scale, 0.0).astype(jnp.float32)
                hf = jnp.bitwise_and(iv, 1).astype(jnp.float32)
                shi16 = sv * hf
                slo16 = sv - shi16

                def row_body(r, carry2):
                    rsel = jnp.full((LANES, 1), r, jnp.int32)
                    dn = lax.GatherDimensionNumbers(
                        offset_dims=(), collapsed_slice_dims=(0,),
                        start_index_map=(0,))
                    pib = lax.GatherScatterMode.PROMISE_IN_BOUNDS
                    slo = lax.gather(slo16, rsel, dn, (1,), mode=pib)
                    shi = lax.gather(shi16, rsel, dn, (1,), mode=pib)
                    row = q * LANES + r
                    prow = lax.div(row, 2)
                    pcol = lax.mul(lax.rem(row, 2), 2 * D // 2)
                    for c in range(D // LANES):
                        lo = rows_v[b, row, pl.ds(c * LANES, LANES)]
                        hi = rows_v[b, row, pl.ds(D + c * LANES, LANES)]
                        out_v[b, prow, pl.ds(pcol + c * LANES, LANES)] = (
                            lo * slo + hi * shi)
                    return carry2
                lax.fori_loop(0, LANES, row_body, 0)
                return carry
            lax.fori_loop(0, granule // LANES, grp_body, 0)

        pltpu.sync_copy(idx_hbm.at[wid], idx_v)

        for b in range(nbuf):
            fire_gather(b, b)

        n_outer = n_gran // nbuf

        def outer(go, carry):
            for b in range(nbuf):
                g = go * nbuf + b
                wait_gather(b)
                compute(g, b)
                fire_scatter(g, b)
            for b in range(nbuf):
                @pl.when(go < n_outer - 1)
                def _():
                    wait_scatter(b)
                    fire_gather((go + 1) * nbuf + b, b)
            return carry

        lax.fori_loop(0, n_outer, outer, 0)

        for b in range(nbuf):
            wait_scatter(b)

    return lookup(idx, table2)


def kernel(input_sequence, table):
    B, S = input_sequence.shape
    V, D = table.shape
    N = B * S
    n_workers = 32
    granule = 128
    nbuf = 4
    idx = input_sequence.reshape(N).astype(jnp.int32)
    idx = idx.reshape(n_workers, N // (n_workers * granule), granule)
    table2 = table.reshape(V // 2, 2 * D)
    out = _sc_embed(idx, table2, granule, nbuf)
    return out.reshape(B, S, D)


# speedup vs baseline: 1.5705x; 1.5705x over previous
import functools
import math

import jax
import jax.numpy as jnp
from jax import lax
from jax.experimental import pallas as pl
from jax.experimental.pallas import tpu as pltpu
from jax.experimental.pallas import tpu_sc as plsc

PAD_IDX = 0
LANES = 16


@functools.partial(jax.jit, static_argnames=("n_workers", "granule", "nbuf"))
def _sc_embed(idx, table, n_workers, granule, nbuf):
    N = idx.shape[0] * idx.shape[1] * idx.shape[2]
    n_gran = idx.shape[1]
    D = table.shape[1]
    scale = math.sqrt(float(D))
    n_per_w = n_gran * granule
    mesh = plsc.VectorSubcoreMesh(core_axis_name="c", subcore_axis_name="s")
    info = plsc.get_sparse_core_info()
    nc = info.num_cores

    @functools.partial(
        pl.kernel,
        mesh=mesh,
        out_type=jax.ShapeDtypeStruct((N, D), jnp.float32),
        compiler_params=pltpu.CompilerParams(use_tc_tiling_on_sc=False),
        scratch_types=[
            pltpu.VMEM((n_gran, granule), jnp.int32),
            pltpu.VMEM((nbuf, granule, D), jnp.float32),
            pltpu.SemaphoreType.DMA((nbuf,)),
            pltpu.SemaphoreType.DMA((nbuf,)),
        ],
    )
    def lookup(idx_hbm, tbl_hbm, out_hbm, idx_v, rows_v, gsem, osem):
        wid = lax.axis_index("s") * nc + lax.axis_index("c")
        wbase = wid * n_per_w

        def fire_gather(g, b):
            pltpu.make_async_copy(
                tbl_hbm.at[idx_v.at[g]], rows_v.at[b], gsem.at[b]
            ).start()

        def wait_gather(b):
            pltpu.make_async_copy(
                tbl_hbm.at[idx_v.at[0]], rows_v.at[b], gsem.at[b]
            ).wait()

        def fire_scatter(g, b):
            pltpu.make_async_copy(
                rows_v.at[b], out_hbm.at[pl.ds(wbase + g * granule, granule)],
                osem.at[b],
            ).start()

        def wait_scatter(b):
            pltpu.make_async_copy(
                rows_v.at[b], out_hbm.at[pl.ds(wbase, granule)], osem.at[b]
            ).wait()

        def compute(g, b):
            def grp_body(q, carry):
                iv = idx_v[g, pl.ds(q * LANES, LANES)]
                sv = jnp.where(iv != PAD_IDX, scale, 0.0).astype(jnp.float32)

                def row_body(r, carry2):
                    splat = lax.gather(
                        sv, jnp.full((LANES, 1), r, jnp.int32),
                        lax.GatherDimensionNumbers(
                            offset_dims=(), collapsed_slice_dims=(0,),
                            start_index_map=(0,)),
                        (1,), mode=lax.GatherScatterMode.PROMISE_IN_BOUNDS)
                    row = q * LANES + r
                    for c in range(D // LANES):
                        sl = pl.ds(c * LANES, LANES)
                        rows_v[b, row, sl] = rows_v[b, row, sl] * splat
                    return carry2
                lax.fori_loop(0, LANES, row_body, 0, unroll=4)
                return carry
            lax.fori_loop(0, granule // LANES, grp_body, 0)

        pltpu.sync_copy(idx_hbm.at[wid], idx_v)

        for b in range(nbuf):
            fire_gather(b, b)

        n_outer = n_gran // nbuf

        def outer(go, carry):
            for b in range(nbuf):
                g = go * nbuf + b
                wait_gather(b)
                compute(g, b)
                fire_scatter(g, b)
            for b in range(nbuf):
                @pl.when(go < n_outer - 1)
                def _():
                    wait_scatter(b)
                    fire_gather((go + 1) * nbuf + b, b)
            return carry

        lax.fori_loop(0, n_outer, outer, 0)

        for b in range(nbuf):
            wait_scatter(b)

    return lookup(idx, table)


def kernel(input_sequence, table):
    B, S = input_sequence.shape
    D = table.shape[1]
    N = B * S
    n_workers = 32
    granule = 256
    nbuf = 4
    idx = input_sequence.reshape(N).astype(jnp.int32)
    idx = idx.reshape(n_workers, N // (n_workers * granule), granule)
    out = _sc_embed(idx, table, n_workers, granule, nbuf)
    return out.reshape(B, S, D)


# speedup vs baseline: 1.5788x; 1.0053x over previous
import functools
import math

import jax
import jax.numpy as jnp
from jax import lax
from jax.experimental import pallas as pl
from jax.experimental.pallas import tpu as pltpu
from jax.experimental.pallas import tpu_sc as plsc

PAD_IDX = 0
LANES = 16


@functools.partial(jax.jit, static_argnames=("n_workers", "granule", "nbuf"))
def _sc_embed(idx, table, n_workers, granule, nbuf):
    N = idx.shape[0] * idx.shape[1] * idx.shape[2]
    n_gran = idx.shape[1]
    D = table.shape[1]
    scale = math.sqrt(float(D))
    n_per_w = n_gran * granule
    assert n_gran % nbuf == 0
    mesh = plsc.VectorSubcoreMesh(core_axis_name="c", subcore_axis_name="s")
    info = plsc.get_sparse_core_info()
    nc = info.num_cores

    @functools.partial(
        pl.kernel,
        mesh=mesh,
        out_type=jax.ShapeDtypeStruct((N, D), jnp.float32),
        compiler_params=pltpu.CompilerParams(use_tc_tiling_on_sc=False),
        scratch_types=[
            pltpu.VMEM((n_gran, granule), jnp.int32),
            pltpu.VMEM((nbuf, granule, D), jnp.float32),
            pltpu.SemaphoreType.DMA((nbuf,)),
            pltpu.SemaphoreType.DMA((nbuf,)),
        ],
    )
    def lookup(idx_hbm, tbl_hbm, out_hbm, idx_v, rows_v, gsem, osem):
        wid = lax.axis_index("s") * nc + lax.axis_index("c")
        wbase = wid * n_per_w

        def fire_gather(g, b):
            pltpu.make_async_copy(
                tbl_hbm.at[idx_v.at[g]], rows_v.at[b], gsem.at[b]
            ).start()

        def wait_gather(b):
            pltpu.make_async_copy(
                tbl_hbm.at[idx_v.at[0]], rows_v.at[b], gsem.at[b]
            ).wait()

        def fire_scatter(g, b):
            pltpu.make_async_copy(
                rows_v.at[b], out_hbm.at[pl.ds(wbase + g * granule, granule)],
                osem.at[b],
            ).start()

        def wait_scatter(b):
            pltpu.make_async_copy(
                rows_v.at[b], out_hbm.at[pl.ds(wbase, granule)], osem.at[b]
            ).wait()

        def compute(g, b):
            def grp_body(q, carry):
                iv = idx_v[g, pl.ds(q * LANES, LANES)]
                sv = jnp.where(iv != PAD_IDX, scale, 0.0).astype(jnp.float32)

                def row_body(r, carry2):
                    splat = lax.gather(
                        sv, jnp.full((LANES, 1), r, jnp.int32),
                        lax.GatherDimensionNumbers(
                            offset_dims=(), collapsed_slice_dims=(0,),
                            start_index_map=(0,)),
                        (1,), mode=lax.GatherScatterMode.PROMISE_IN_BOUNDS)
                    row = q * LANES + r
                    for c in range(D // LANES):
                        sl = pl.ds(c * LANES, LANES)
                        rows_v[b, row, sl] = rows_v[b, row, sl] * splat
                    return carry2
                lax.fori_loop(0, LANES, row_body, 0, unroll=4)
                return carry
            lax.fori_loop(0, granule // LANES, grp_body, 0, unroll=2)

        pltpu.sync_copy(idx_hbm.at[wid], idx_v)

        for b in range(nbuf):
            fire_gather(b, b)

        n_outer = n_gran // nbuf

        def outer(go, carry):
            for b in range(nbuf):
                g = go * nbuf + b
                wait_gather(b)
                compute(g, b)
                fire_scatter(g, b)
            for b in range(nbuf):
                @pl.when(go < n_outer - 1)
                def _():
                    wait_scatter(b)
                    fire_gather((go + 1) * nbuf + b, b)
            return carry

        lax.fori_loop(0, n_outer, outer, 0)

        for b in range(nbuf):
            wait_scatter(b)

    return lookup(idx, table)


def kernel(input_sequence, table):
    B, S = input_sequence.shape
    D = table.shape[1]
    N = B * S
    n_workers = 32
    granule = 512
    nbuf = 2
    idx = input_sequence.reshape(N).astype(jnp.int32)
    idx = idx.reshape(n_workers, N // (n_workers * granule), granule)
    out = _sc_embed(idx, table, n_workers, granule, nbuf)
    return out.reshape(B, S, D)
